# baseline (speedup 1.0000x reference)
import functools

import jax
import jax.numpy as jnp
from jax import lax
from jax.experimental import pallas as pl
from jax.experimental.pallas import tpu as pltpu
from jax.experimental.pallas import tpu_sc as plsc

E = 768
POOL = 100
K = 5
PLEN = 5
H = 12
HD = 64
L2 = 24
B = 4
SEQ = 2048
SEQ_CHUNK = 256
NCH = SEQ // SEQ_CHUNK
ROW = 320
RPE = (PLEN * H * HD) // ROW
RPB = 2 * K * RPE
NEG = -3.0e38


def _phase1_body(x_ref, skey_ref, mkey_ref,
                 ssim_ref, msim_ref, sidx_ref, midx_ref, rsum_ref,
                 pidx_ref):
    mean = jnp.sum(x_ref[...], axis=1) * (1.0 / SEQ)
    sq = jnp.sum(mean * mean, axis=-1, keepdims=True)
    xn = mean * lax.rsqrt(jnp.maximum(sq, 1e-12))

    kcol = lax.broadcasted_iota(jnp.int32, (B, K), 1)
    lane16 = lax.broadcasted_iota(jnp.int32, (B, 16), 1)

    def pool_topk(key_ref, sim_ref, idx_ref, pool_id, comb):
        kv = key_ref[...]
        inv = lax.rsqrt(jnp.maximum(jnp.sum(kv * kv, axis=-1,
                                            keepdims=True), 1e-12))
        kn = kv * inv
        sim = lax.dot_general(xn.astype(jnp.bfloat16),
                              kn.astype(jnp.bfloat16),
                              (((1,), (1,)), ((), ())),
                              preferred_element_type=jnp.float32)
        sim_ref[...] = sim
        iota = lax.broadcasted_iota(jnp.int32, (B, POOL), 1)
        cur = sim
        idxs = jnp.zeros((B, K), jnp.int32)
        tot = jnp.float32(0.0)
        for t in range(K):
            mx = jnp.max(cur, axis=1, keepdims=True)
            pos = jnp.min(jnp.where(cur == mx, iota, POOL),
                          axis=1, keepdims=True)
            tot = tot + jnp.sum(mx)
            idxs = jnp.where(kcol == t, pos, idxs)
            comb = jnp.where((lane16 // 8 == pool_id) & (lane16 % 8 == t),
                             pos, comb)
            cur = jnp.where(iota == pos, NEG, cur)
        idx_ref[...] = idxs
        return tot / B, comb

    comb = jnp.zeros((B, 16), jnp.int32)
    s_rs, comb = pool_topk(skey_ref, ssim_ref, sidx_ref, 0, comb)
    m_rs, comb = pool_topk(mkey_ref, msim_ref, midx_ref, 1, comb)
    pidx_ref[...] = comb
    two = lax.broadcasted_iota(jnp.int32, (1, 2), 1)
    rsum_ref[...] = jnp.where(two == 0, s_rs, m_rs)


def _phase1(x_embed, s_prompt_key, m_prompt_key):
    out_shapes = (
        jax.ShapeDtypeStruct((B, POOL), jnp.float32),
        jax.ShapeDtypeStruct((B, POOL), jnp.float32),
        jax.ShapeDtypeStruct((B, K), jnp.int32),
        jax.ShapeDtypeStruct((B, K), jnp.int32),
        jax.ShapeDtypeStruct((1, 2), jnp.float32),
        jax.ShapeDtypeStruct((B, 16), jnp.int32),
    )
    full = lambda shape: pl.BlockSpec(shape, lambda: (0,) * len(shape))
    return pl.pallas_call(
        _phase1_body,
        in_specs=[
            full((B, SEQ, E)),
            full((POOL, E)),
            full((POOL, E)),
        ],
        out_specs=(
            full((B, POOL)), full((B, POOL)),
            full((B, K)), full((B, K)), full((1, 2)), full((B, 16)),
        ),
        out_shape=out_shapes,
    )(x_embed, s_prompt_key, m_prompt_key)


def _gather_body(sp_ref, mp_ref, pidx_ref, sout_ref, mout_ref):
    pid = pidx_ref[...]
    piota = lax.broadcasted_iota(jnp.int32, (B, K, POOL), 2)

    def sel(tab_ref, base):
        oh = (piota == pid[:, base:base + K, None]).astype(jnp.float32)
        oh2 = oh.reshape(B * K, POOL)
        tab = tab_ref[...].reshape(H * HD, POOL)
        return lax.dot_general(oh2, tab, (((1,), (1,)), ((), ())),
                               precision=lax.Precision.HIGHEST,
                               preferred_element_type=jnp.float32)

    sout_ref[...] = sel(sp_ref, 0)[None, :, None, None]
    mout_ref[...] = sel(mp_ref, 8)[None, :, None, None]


def _phase2(s_prompt, m_prompt, pidx):
    sp_t = jnp.transpose(s_prompt, (0, 2, 3, 4, 1))
    mp_t = jnp.transpose(m_prompt, (0, 2, 3, 4, 1))
    blk5 = pl.BlockSpec((1, 1, H, HD, POOL), lambda l, i: (l, i, 0, 0, 0))
    s_out, m_out = pl.pallas_call(
        _gather_body,
        grid=(L2, PLEN),
        in_specs=[
            blk5, blk5,
            pl.BlockSpec((B, 16), lambda l, i: (0, 0)),
        ],
        out_specs=(
            pl.BlockSpec((1, B * K, 1, 1, H * HD),
                         lambda l, i: (l, 0, i, 0, 0)),
            pl.BlockSpec((1, B * K, 1, 1, H * HD),
                         lambda l, i: (l, 0, i, 0, 0)),
        ),
        out_shape=(
            jax.ShapeDtypeStruct((L2, B * K, PLEN, 1, H * HD), jnp.float32),
            jax.ShapeDtypeStruct((L2, B * K, PLEN, 1, H * HD), jnp.float32),
        ),
    )(sp_t, mp_t, pidx)
    s_bp = s_out.reshape(L2, B, H, K * PLEN, HD)
    m_bp = m_out.reshape(L2, B, H, K * PLEN, HD)
    return jnp.concatenate([s_bp, m_bp], axis=3)


def kernel(x_embed, s_prompt, m_prompt, s_prompt_key, m_prompt_key):
    s_sim, m_sim, s_idx, m_idx, rsum, pidx = _phase1(
        x_embed, s_prompt_key, m_prompt_key)
    batched_prompt = _phase2(s_prompt, m_prompt, pidx)
    s_reduce = rsum[0, 0].reshape(())
    m_reduce = rsum[0, 1].reshape(())
    return (batched_prompt, s_sim, m_sim, s_reduce, m_reduce, s_idx, m_idx)


# speedup vs baseline: 1.6810x; 1.6810x over previous
import functools

import jax
import jax.numpy as jnp
from jax import lax
from jax.experimental import pallas as pl
from jax.experimental.pallas import tpu as pltpu
from jax.experimental.pallas import tpu_sc as plsc

E = 768
POOL = 100
K = 5
PLEN = 5
H = 12
HD = 64
L2 = 24
B = 4
SEQ = 2048
SEQ_CHUNK = 256
NCH = SEQ // SEQ_CHUNK
ROW = 320
RPE = (PLEN * H * HD) // ROW
RPB = 2 * K * RPE
NEG = -3.0e38


def _phase1_body(x_ref, skey_ref, mkey_ref,
                 ssim_ref, msim_ref, sidx_ref, midx_ref, rsum_ref,
                 pidx_ref):
    mean = jnp.sum(x_ref[...], axis=1) * (1.0 / SEQ)
    sq = jnp.sum(mean * mean, axis=-1, keepdims=True)
    xn = mean * lax.rsqrt(jnp.maximum(sq, 1e-12))

    kcol = lax.broadcasted_iota(jnp.int32, (B, K), 1)
    lane16 = lax.broadcasted_iota(jnp.int32, (B, 16), 1)

    def pool_topk(key_ref, sim_ref, idx_ref, pool_id, comb):
        kv = key_ref[...]
        inv = lax.rsqrt(jnp.maximum(jnp.sum(kv * kv, axis=-1,
                                            keepdims=True), 1e-12))
        kn = kv * inv
        sim = lax.dot_general(xn.astype(jnp.bfloat16),
                              kn.astype(jnp.bfloat16),
                              (((1,), (1,)), ((), ())),
                              preferred_element_type=jnp.float32)
        sim_ref[...] = sim
        iota = lax.broadcasted_iota(jnp.int32, (B, POOL), 1)
        cur = sim
        idxs = jnp.zeros((B, K), jnp.int32)
        tot = jnp.float32(0.0)
        for t in range(K):
            mx = jnp.max(cur, axis=1, keepdims=True)
            pos = jnp.min(jnp.where(cur == mx, iota, POOL),
                          axis=1, keepdims=True)
            tot = tot + jnp.sum(mx)
            idxs = jnp.where(kcol == t, pos, idxs)
            comb = jnp.where((lane16 // 8 == pool_id) & (lane16 % 8 == t),
                             pos, comb)
            cur = jnp.where(iota == pos, NEG, cur)
        idx_ref[...] = idxs
        return tot / B, comb

    comb = jnp.zeros((B, 16), jnp.int32)
    s_rs, comb = pool_topk(skey_ref, ssim_ref, sidx_ref, 0, comb)
    m_rs, comb = pool_topk(mkey_ref, msim_ref, midx_ref, 1, comb)
    pidx_ref[...] = comb
    two = lax.broadcasted_iota(jnp.int32, (1, 2), 1)
    rsum_ref[...] = jnp.where(two == 0, s_rs, m_rs)


def _phase1(x_embed, s_prompt_key, m_prompt_key):
    out_shapes = (
        jax.ShapeDtypeStruct((B, POOL), jnp.float32),
        jax.ShapeDtypeStruct((B, POOL), jnp.float32),
        jax.ShapeDtypeStruct((B, K), jnp.int32),
        jax.ShapeDtypeStruct((B, K), jnp.int32),
        jax.ShapeDtypeStruct((1, 2), jnp.float32),
        jax.ShapeDtypeStruct((B, 16), jnp.int32),
    )
    full = lambda shape: pl.BlockSpec(shape, lambda: (0,) * len(shape))
    return pl.pallas_call(
        _phase1_body,
        in_specs=[
            full((B, SEQ, E)),
            full((POOL, E)),
            full((POOL, E)),
        ],
        out_specs=(
            full((B, POOL)), full((B, POOL)),
            full((B, K)), full((B, K)), full((1, 2)), full((B, 16)),
        ),
        out_shape=out_shapes,
    )(x_embed, s_prompt_key, m_prompt_key)


def _gather_body(sp_ref, mp_ref, pidx_ref, sout_ref, mout_ref):
    pid = pidx_ref[...]
    piota = lax.broadcasted_iota(jnp.int32, (B, K, POOL), 2)

    def sel(tab_ref, base):
        oh = (piota == pid[:, base:base + K, None]).astype(jnp.float32)
        oh2 = oh.reshape(B * K, POOL)
        tab = tab_ref[...].reshape(PLEN * H * HD, POOL)
        ohb = oh2.astype(jnp.bfloat16)
        hi = tab.astype(jnp.bfloat16)
        r1 = tab - hi.astype(jnp.float32)
        mid = r1.astype(jnp.bfloat16)
        lo = (r1 - mid.astype(jnp.float32)).astype(jnp.bfloat16)
        dims = (((1,), (1,)), ((), ()))
        acc = lax.dot_general(ohb, hi, dims,
                              preferred_element_type=jnp.float32)
        acc = acc + lax.dot_general(ohb, mid, dims,
                                    preferred_element_type=jnp.float32)
        return acc + lax.dot_general(ohb, lo, dims,
                                     preferred_element_type=jnp.float32)

    sout_ref[...] = sel(sp_ref, 0)[None]
    mout_ref[...] = sel(mp_ref, 8)[None]


def _phase2(s_prompt, m_prompt, pidx):
    sp_t = jnp.transpose(s_prompt, (0, 2, 3, 4, 1))
    mp_t = jnp.transpose(m_prompt, (0, 2, 3, 4, 1))
    ent = PLEN * H * HD
    full5 = pl.BlockSpec((1, PLEN, H, HD, POOL), lambda l: (l, 0, 0, 0, 0))
    s_out, m_out = pl.pallas_call(
        _gather_body,
        grid=(L2,),
        in_specs=[
            full5, full5,
            pl.BlockSpec((B, 16), lambda l: (0, 0)),
        ],
        out_specs=(
            pl.BlockSpec((1, B * K, ent), lambda l: (l, 0, 0)),
            pl.BlockSpec((1, B * K, ent), lambda l: (l, 0, 0)),
        ),
        out_shape=(
            jax.ShapeDtypeStruct((L2, B * K, ent), jnp.float32),
            jax.ShapeDtypeStruct((L2, B * K, ent), jnp.float32),
        ),
    )(sp_t, mp_t, pidx)
    s_bp = s_out.reshape(L2, B, H, K * PLEN, HD)
    m_bp = m_out.reshape(L2, B, H, K * PLEN, HD)
    return jnp.concatenate([s_bp, m_bp], axis=3)


def kernel(x_embed, s_prompt, m_prompt, s_prompt_key, m_prompt_key):
    s_sim, m_sim, s_idx, m_idx, rsum, pidx = _phase1(
        x_embed, s_prompt_key, m_prompt_key)
    batched_prompt = _phase2(s_prompt, m_prompt, pidx)
    s_reduce = rsum[0, 0].reshape(())
    m_reduce = rsum[0, 1].reshape(())
    return (batched_prompt, s_sim, m_sim, s_reduce, m_reduce, s_idx, m_idx)
